# Initial kernel scaffold; baseline (speedup 1.0000x reference)
import functools

import jax
import jax.numpy as jnp
from jax import lax
from jax.experimental import pallas as pl
from jax.experimental.pallas import tpu as pltpu
import jax.experimental.pallas.tpu_sc as plsc


def _floor_i32(v):
    t = v.astype(jnp.int32)
    tf = t.astype(jnp.float32)
    return jnp.where(tf > v, t - 1, t)


def kernel(features, grid):
    B, H, W, C = features.shape
    P = grid.shape[1]
    N = B * P
    L = 16

    feat = features.reshape(B * H * W, C).astype(jnp.float32)
    gy = grid[..., 1].reshape(N).astype(jnp.float32)
    gx = grid[..., 0].reshape(N).astype(jnp.float32)

    mesh = plsc.VectorSubcoreMesh(core_axis_name="c", subcore_axis_name="s")
    NW = mesh.num_cores * mesh.num_subcores
    n_per_w = N // NW
    PTS = 32
    n_it = n_per_w // PTS

    @functools.partial(
        pl.kernel,
        mesh=mesh,
        out_type=jax.ShapeDtypeStruct((N, C), jnp.float32),
        scratch_types=[
            pltpu.VMEM((n_per_w,), jnp.float32),
            pltpu.VMEM((n_per_w,), jnp.float32),
            [pltpu.VMEM((PTS,), jnp.int32) for _ in range(4)],
            [pltpu.VMEM((PTS,), jnp.float32) for _ in range(4)],
            [pltpu.VMEM((PTS, C), jnp.float32) for _ in range(4)],
            pltpu.VMEM((PTS, C), jnp.float32),
            pltpu.SemaphoreType.DMA,
        ],
    )
    def run(feat_hbm, gy_hbm, gx_hbm, out_hbm,
            gy_v, gx_v, idx_vs, w_vs, row_vs, ob_v, sem):
        cid = lax.axis_index("c")
        sid = lax.axis_index("s")
        wid = sid * mesh.num_cores + cid
        base = wid * n_per_w
        boff = (base // P) * (H * W)

        pltpu.sync_copy(gy_hbm.at[pl.ds(base, n_per_w)], gy_v)
        pltpu.sync_copy(gx_hbm.at[pl.ds(base, n_per_w)], gx_v)

        corners = ((0, 0), (1, 0), (0, 1), (1, 1))

        def it_body(it, carry):
            for sub in range(PTS // L):
                off = it * PTS + sub * L
                y = gy_v[pl.ds(off, L)] * float(H) - 0.5
                x = gx_v[pl.ds(off, L)] * float(W) - 0.5
                yi = _floor_i32(y)
                xi = _floor_i32(x)
                fy = y - yi.astype(jnp.float32)
                fx = x - xi.astype(jnp.float32)
                wy = (1.0 - fy, fy)
                wx = (1.0 - fx, fx)
                for ci, (dy, dx) in enumerate(corners):
                    yc = yi + dy
                    xc = xi + dx
                    valid = ((yc >= 0) & (yc < H) & (xc >= 0) & (xc < W))
                    ycl = jnp.clip(yc, 0, H - 1)
                    xcl = jnp.clip(xc, 0, W - 1)
                    idx_vs[ci][pl.ds(sub * L, L)] = boff + ycl * W + xcl
                    w = wy[dy] * wx[dx]
                    w_vs[ci][pl.ds(sub * L, L)] = jnp.where(valid, w, 0.0)

            cps = [pltpu.async_copy(feat_hbm.at[idx_vs[ci]], row_vs[ci], sem)
                   for ci in range(4)]
            for cp in cps:
                cp.wait()

            def pt_body(j, c2):
                w0 = w_vs[0][j]
                w1 = w_vs[1][j]
                w2 = w_vs[2][j]
                w3 = w_vs[3][j]
                for cb in range(C // L):
                    s = pl.ds(cb * L, L)
                    ob_v[j, s] = (w0 * row_vs[0][j, s] + w1 * row_vs[1][j, s]
                                  + w2 * row_vs[2][j, s] + w3 * row_vs[3][j, s])
                return c2

            lax.fori_loop(0, PTS, pt_body, 0)
            pltpu.sync_copy(ob_v, out_hbm.at[pl.ds(base + it * PTS, PTS)])
            return carry

        lax.fori_loop(0, n_it, it_body, 0)

    out = run(feat, gy, gx)
    return out.reshape(B, P, C).astype(features.dtype)


# speedup vs baseline: 1.4832x; 1.4832x over previous
import functools

import jax
import jax.numpy as jnp
from jax import lax
from jax.experimental import pallas as pl
from jax.experimental.pallas import tpu as pltpu
import jax.experimental.pallas.tpu_sc as plsc


def _floor_i32(v):
    t = v.astype(jnp.int32)
    tf = t.astype(jnp.float32)
    return jnp.where(tf > v, t - 1, t)


def kernel(features, grid):
    B, H, W, C = features.shape
    P = grid.shape[1]
    N = B * P
    L = 16

    feat = features.reshape(B * H * W, C).astype(jnp.float32)
    gy = grid[..., 1].reshape(N).astype(jnp.float32)
    gx = grid[..., 0].reshape(N).astype(jnp.float32)

    mesh = plsc.VectorSubcoreMesh(core_axis_name="c", subcore_axis_name="s")
    NW = mesh.num_cores * mesh.num_subcores
    n_per_w = N // NW
    PTS = 32
    n_it = n_per_w // PTS

    @functools.partial(
        pl.kernel,
        mesh=mesh,
        out_type=jax.ShapeDtypeStruct((N, C), jnp.float32),
        scratch_types=[
            pltpu.VMEM((n_per_w,), jnp.float32),
            pltpu.VMEM((n_per_w,), jnp.float32),
            [pltpu.VMEM((PTS,), jnp.int32) for _ in range(4)],
            [pltpu.VMEM((PTS + L,), jnp.float32) for _ in range(4)],
            [pltpu.VMEM((PTS, C), jnp.float32) for _ in range(4)],
            pltpu.VMEM((PTS, C), jnp.float32),
            pltpu.SemaphoreType.DMA,
        ],
    )
    def run(feat_hbm, gy_hbm, gx_hbm, out_hbm,
            gy_v, gx_v, idx_vs, w_vs, row_vs, ob_v, sem):
        cid = lax.axis_index("c")
        sid = lax.axis_index("s")
        wid = sid * mesh.num_cores + cid
        base = wid * n_per_w
        boff = (base // P) * (H * W)

        pltpu.sync_copy(gy_hbm.at[pl.ds(base, n_per_w)], gy_v)
        pltpu.sync_copy(gx_hbm.at[pl.ds(base, n_per_w)], gx_v)

        corners = ((0, 0), (1, 0), (0, 1), (1, 1))

        def it_body(it, carry):
            for sub in range(PTS // L):
                off = it * PTS + sub * L
                y = gy_v[pl.ds(off, L)] * float(H) - 0.5
                x = gx_v[pl.ds(off, L)] * float(W) - 0.5
                yi = _floor_i32(y)
                xi = _floor_i32(x)
                fy = y - yi.astype(jnp.float32)
                fx = x - xi.astype(jnp.float32)
                wy = (1.0 - fy, fy)
                wx = (1.0 - fx, fx)
                for ci, (dy, dx) in enumerate(corners):
                    yc = yi + dy
                    xc = xi + dx
                    valid = ((yc >= 0) & (yc < H) & (xc >= 0) & (xc < W))
                    ycl = jnp.clip(yc, 0, H - 1)
                    xcl = jnp.clip(xc, 0, W - 1)
                    idx_vs[ci][pl.ds(sub * L, L)] = boff + ycl * W + xcl
                    w = wy[dy] * wx[dx]
                    w_vs[ci][pl.ds(sub * L, L)] = jnp.where(valid, w, 0.0)

            cps = [pltpu.async_copy(feat_hbm.at[idx_vs[ci]], row_vs[ci], sem)
                   for ci in range(4)]
            for cp in cps:
                cp.wait()

            def pt_body(j, c2):
                w0 = w_vs[0][pl.ds(j, L)][0]
                w1 = w_vs[1][pl.ds(j, L)][0]
                w2 = w_vs[2][pl.ds(j, L)][0]
                w3 = w_vs[3][pl.ds(j, L)][0]
                for cb in range(C // L):
                    s = pl.ds(cb * L, L)
                    ob_v[j, s] = (w0 * row_vs[0][j, s] + w1 * row_vs[1][j, s]
                                  + w2 * row_vs[2][j, s] + w3 * row_vs[3][j, s])
                return c2

            lax.fori_loop(0, PTS, pt_body, 0)
            pltpu.sync_copy(ob_v, out_hbm.at[pl.ds(base + it * PTS, PTS)])
            return carry

        lax.fori_loop(0, n_it, it_body, 0)

    out = run(feat, gy, gx)
    return out.reshape(B, P, C).astype(features.dtype)


# speedup vs baseline: 2.6443x; 1.7828x over previous
import functools

import jax
import jax.numpy as jnp
from jax import lax
from jax.experimental import pallas as pl
from jax.experimental.pallas import tpu as pltpu
import jax.experimental.pallas.tpu_sc as plsc


def _floor_i32(v):
    t = v.astype(jnp.int32)
    tf = t.astype(jnp.float32)
    return jnp.where(tf > v, t - 1, t)


def kernel(features, grid):
    B, H, W, C = features.shape
    P = grid.shape[1]
    N = B * P
    L = 16

    feat = features.reshape(B * H * W, C).astype(jnp.float32)
    gy = grid[..., 1].reshape(N).astype(jnp.float32)
    gx = grid[..., 0].reshape(N).astype(jnp.float32)

    mesh = plsc.VectorSubcoreMesh(core_axis_name="c", subcore_axis_name="s")
    NW = mesh.num_cores * mesh.num_subcores
    n_per_w = N // NW
    PTS = 32
    n_it = n_per_w // PTS
    NB = 2

    @functools.partial(
        pl.kernel,
        mesh=mesh,
        out_type=jax.ShapeDtypeStruct((N, C), jnp.float32),
        scratch_types=[
            pltpu.VMEM((n_per_w,), jnp.float32),
            pltpu.VMEM((n_per_w,), jnp.float32),
            [[pltpu.VMEM((PTS,), jnp.int32) for _ in range(4)]
             for _ in range(NB)],
            [[pltpu.VMEM((PTS + L,), jnp.float32) for _ in range(4)]
             for _ in range(NB)],
            [[pltpu.VMEM((PTS, C), jnp.float32) for _ in range(4)]
             for _ in range(NB)],
            [pltpu.VMEM((PTS, C), jnp.float32) for _ in range(NB)],
            [pltpu.SemaphoreType.DMA for _ in range(NB)],
            [pltpu.SemaphoreType.DMA for _ in range(NB)],
        ],
    )
    def run(feat_hbm, gy_hbm, gx_hbm, out_hbm,
            gy_v, gx_v, idx_vs, w_vs, row_vs, ob_vs, gsems, osems):
        cid = lax.axis_index("c")
        sid = lax.axis_index("s")
        wid = sid * mesh.num_cores + cid
        base = wid * n_per_w
        boff = (base // P) * (H * W)

        pltpu.sync_copy(gy_hbm.at[pl.ds(base, n_per_w)], gy_v)
        pltpu.sync_copy(gx_hbm.at[pl.ds(base, n_per_w)], gx_v)

        corners = ((0, 0), (1, 0), (0, 1), (1, 1))

        def fire(it, s):
            for sub in range(PTS // L):
                off = it * PTS + sub * L
                y = gy_v[pl.ds(off, L)] * float(H) - 0.5
                x = gx_v[pl.ds(off, L)] * float(W) - 0.5
                yi = _floor_i32(y)
                xi = _floor_i32(x)
                fy = y - yi.astype(jnp.float32)
                fx = x - xi.astype(jnp.float32)
                wy = (1.0 - fy, fy)
                wx = (1.0 - fx, fx)
                for ci, (dy, dx) in enumerate(corners):
                    yc = yi + dy
                    xc = xi + dx
                    valid = ((yc >= 0) & (yc < H) & (xc >= 0) & (xc < W))
                    ycl = jnp.clip(yc, 0, H - 1)
                    xcl = jnp.clip(xc, 0, W - 1)
                    idx_vs[s][ci][pl.ds(sub * L, L)] = boff + ycl * W + xcl
                    w = wy[dy] * wx[dx]
                    w_vs[s][ci][pl.ds(sub * L, L)] = jnp.where(valid, w, 0.0)
            for ci in range(4):
                pltpu.async_copy(feat_hbm.at[idx_vs[s][ci]], row_vs[s][ci],
                                 gsems[s])

        def consume(it, s, first):
            for ci in range(4):
                pltpu.make_async_copy(feat_hbm.at[idx_vs[s][ci]],
                                      row_vs[s][ci], gsems[s]).wait()
            if not first:
                pltpu.make_async_copy(
                    ob_vs[s], out_hbm.at[pl.ds(base, PTS)], osems[s]).wait()

            def pt_body(j, c2):
                w0 = w_vs[s][0][pl.ds(j, L)][0]
                w1 = w_vs[s][1][pl.ds(j, L)][0]
                w2 = w_vs[s][2][pl.ds(j, L)][0]
                w3 = w_vs[s][3][pl.ds(j, L)][0]
                for cb in range(C // L):
                    sl = pl.ds(cb * L, L)
                    ob_vs[s][j, sl] = (
                        w0 * row_vs[s][0][j, sl] + w1 * row_vs[s][1][j, sl]
                        + w2 * row_vs[s][2][j, sl] + w3 * row_vs[s][3][j, sl])
                return c2

            lax.fori_loop(0, PTS, pt_body, 0)
            pltpu.async_copy(ob_vs[s], out_hbm.at[pl.ds(base + it * PTS, PTS)],
                             osems[s])

        fire(0, 0)
        fire(1, 1)

        def it_body(it2, carry):
            it = it2 * NB
            consume(it, 0, False)
            fire(it + 2, 0)
            consume(it + 1, 1, False)
            fire(it + 3, 1)
            return carry

        consume(0, 0, True)
        fire(2, 0)
        consume(1, 1, True)
        fire(3, 1)
        lax.fori_loop(1, n_it // NB - 1, it_body, 0)
        consume(n_it - 2, 0, False)
        consume(n_it - 1, 1, False)
        for s in range(NB):
            pltpu.make_async_copy(
                ob_vs[s], out_hbm.at[pl.ds(base, PTS)], osems[s]).wait()

    out = run(feat, gy, gx)
    return out.reshape(B, P, C).astype(features.dtype)
